# Initial kernel scaffold; baseline (speedup 1.0000x reference)
import jax
import jax.numpy as jnp
from jax.experimental import pallas as pl


def kernel(x, tables):
    raise NotImplementedError("write your pallas kernel here")


# speedup vs baseline: 1.5342x; 1.5342x over previous
import functools

import jax
import jax.numpy as jnp
from jax import lax
from jax.experimental import pallas as pl
from jax.experimental.pallas import tpu as pltpu
from jax.experimental.pallas import tpu_sc as plsc

NUM_FIELDS = 26
FIELD = 100000
TOTAL = NUM_FIELDS * FIELD
DIM = 32

NC = 2
NS = 16
NW = NC * NS

X_CHUNK = 4000
N_CHUNKS = FIELD // X_CHUNK

SUPER = 1024
GPT = SUPER // 128
BASE_ROWS = 81248
N_SUPER = BASE_ROWS // SUPER
REM = BASE_ROWS - N_SUPER * SUPER
REM_LAST = REM + (TOTAL - NW * BASE_ROWS)


def _compact_kernel(x_hbm, idx_hbm, xbuf, idxbuf):
    wid = lax.axis_index("s") * NC + lax.axis_index("c")

    @pl.when(wid < NUM_FIELDS)
    def _():
        field_base = wid * FIELD
        base_vec = jnp.full((16,), field_base, jnp.int32)

        def prefill(k, _):
            idxbuf[pl.ds(k * 16, 16)] = base_vec
            return 0
        lax.fori_loop(0, FIELD // 16, prefill, 0)

        lane = lax.iota(jnp.int32, 16)
        one_vec = jnp.full((16,), 1, jnp.int32)
        zero_vec = jnp.zeros((16,), jnp.int32)

        def chunk_body(c, cursor):
            pltpu.sync_copy(x_hbm.at[pl.ds(field_base + c * X_CHUNK, X_CHUNK)],
                            xbuf)

            def vreg_body(k, cur):
                v = xbuf[pl.ds(k * 16, 16)]
                m = v.astype(jnp.int32) == 1
                mi = jnp.where(m, one_vec, zero_vec)
                incl = plsc.cumsum(mi)
                cur_vec = jnp.full((16,), cur, jnp.int32)
                dest = (cur_vec + incl) - mi
                off = field_base + c * X_CHUNK + k * 16
                pos = jnp.full((16,), off, jnp.int32) + lane
                plsc.store_scatter(idxbuf, [dest], pos, mask=m)
                return cur + jnp.sum(mi)

            return lax.fori_loop(0, X_CHUNK // 16, vreg_body, cursor)

        lax.fori_loop(0, N_CHUNKS, chunk_body, jnp.int32(0))
        pltpu.sync_copy(idxbuf, idx_hbm.at[pl.ds(field_base, FIELD)])


def _gather_kernel(table_hbm, idx_hbm, out_hbm, idxbuf, rowbuf, sems):
    wid = lax.axis_index("s") * NC + lax.axis_index("c")
    base = wid * BASE_ROWS

    def super_body(t, _):
        row0 = base + t * SUPER
        pltpu.sync_copy(idx_hbm.at[pl.ds(row0, SUPER)], idxbuf)
        for j in range(GPT):
            pltpu.async_copy(
                table_hbm.at[idxbuf.at[pl.ds(j * 128, 128)]],
                rowbuf.at[pl.ds(j * 128, 128), :],
                sems)
        for j in range(GPT):
            pltpu.make_async_copy(
                table_hbm.at[idxbuf.at[pl.ds(j * 128, 128)]],
                rowbuf.at[pl.ds(j * 128, 128), :],
                sems).wait()
        pltpu.sync_copy(rowbuf, out_hbm.at[pl.ds(row0, SUPER), :])
        return 0

    lax.fori_loop(0, N_SUPER, super_body, 0)

    n_rem = jnp.where(wid == NW - 1, REM_LAST // 32, REM // 32)

    def rem_body(r, _):
        row0 = base + N_SUPER * SUPER + r * 32
        pltpu.sync_copy(idx_hbm.at[pl.ds(row0, 32)], idxbuf.at[pl.ds(0, 32)])
        pltpu.async_copy(
            table_hbm.at[idxbuf.at[pl.ds(0, 32)]],
            rowbuf.at[pl.ds(0, 32), :],
            sems).wait()
        pltpu.sync_copy(rowbuf.at[pl.ds(0, 32), :],
                        out_hbm.at[pl.ds(row0, 32), :])
        return 0

    lax.fori_loop(0, n_rem, rem_body, 0)


@jax.jit
def kernel(x, tables):
    mesh = plsc.VectorSubcoreMesh(core_axis_name="c", subcore_axis_name="s")

    compact = pl.kernel(
        _compact_kernel,
        mesh=mesh,
        compiler_params=pltpu.CompilerParams(needs_layout_passes=False),
        out_type=jax.ShapeDtypeStruct((TOTAL,), jnp.int32),
        scratch_types=[
            pltpu.VMEM((X_CHUNK,), jnp.float32),
            pltpu.VMEM((FIELD,), jnp.int32),
        ],
    )
    idx = compact(x)

    gather = pl.kernel(
        _gather_kernel,
        mesh=mesh,
        compiler_params=pltpu.CompilerParams(
            needs_layout_passes=False, use_tc_tiling_on_sc=False),
        out_type=jax.ShapeDtypeStruct((TOTAL, DIM), jnp.float32),
        scratch_types=[
            pltpu.VMEM((SUPER,), jnp.int32),
            pltpu.VMEM((SUPER, DIM), jnp.float32),
            pltpu.SemaphoreType.DMA,
        ],
    )
    return gather(tables.reshape(TOTAL, DIM), idx)
